# Initial kernel scaffold; baseline (speedup 1.0000x reference)
import jax
import jax.numpy as jnp
from jax.experimental import pallas as pl


def kernel(xyz, params):
    raise NotImplementedError("write your pallas kernel here")


# speedup vs baseline: 10.0496x; 10.0496x over previous
import jax
import jax.numpy as jnp
from jax.experimental import pallas as pl
from jax.experimental.pallas import tpu as pltpu


def _fps(coords, npoint, n):
    lane = jax.lax.broadcasted_iota(jnp.int32, (1, n), 1)
    col = jax.lax.broadcasted_iota(jnp.int32, (3, npoint), 1)
    rowp = jax.lax.broadcasted_iota(jnp.int32, (npoint, 3), 0)

    def body(i, carry):
        dist, far, new_xyz, nxT = carry
        onehot = (lane == far).astype(jnp.float32)
        cx = jnp.sum(coords[0:1] * onehot, axis=1, keepdims=True)
        cy = jnp.sum(coords[1:2] * onehot, axis=1, keepdims=True)
        cz = jnp.sum(coords[2:3] * onehot, axis=1, keepdims=True)
        centroid = jnp.concatenate([cx, cy, cz], axis=0)
        centroidT = jnp.concatenate([cx, cy, cz], axis=1)
        new_xyz = jnp.where(col == i, centroid, new_xyz)
        nxT = jnp.where(rowp == i, centroidT, nxT)
        d = ((coords[0:1] - cx) ** 2 + (coords[1:2] - cy) ** 2
             + (coords[2:3] - cz) ** 2)
        dist = jnp.minimum(dist, d)
        mx = jnp.max(dist, axis=1, keepdims=True)
        far = jnp.min(jnp.where(dist == mx, lane, n), axis=1, keepdims=True)
        return dist, far, new_xyz, nxT

    dist0 = jnp.full((1, n), 1e10, dtype=jnp.float32)
    far0 = jnp.zeros((1, 1), dtype=jnp.int32)
    nx0 = jnp.zeros((3, npoint), dtype=jnp.float32)
    nxT0 = jnp.zeros((npoint, 3), dtype=jnp.float32)
    _, _, new_xyz, nxT = jax.lax.fori_loop(0, npoint, body,
                                           (dist0, far0, nx0, nxT0))
    return new_xyz, nxT


def _ball_sel(coords, nxT, r2, S, K, n):
    d2 = ((nxT[:, 0:1] - coords[0:1, :]) ** 2
          + (nxT[:, 1:2] - coords[1:2, :]) ** 2
          + (nxT[:, 2:3] - coords[2:3, :]) ** 2)
    inmask = jnp.logical_not(d2 > jnp.float32(r2))
    maskf = inmask.astype(jnp.float32)
    rank = maskf
    s = 1
    while s < n:
        rank = rank + jnp.concatenate(
            [jnp.zeros((S, s), jnp.float32), rank[:, :n - s]], axis=1)
        s *= 2
    count = rank[:, n - 1:n]
    rank3 = rank.reshape(1, S, n)
    mask3 = inmask.reshape(1, S, n)
    kval = jax.lax.broadcasted_iota(jnp.int32, (K, 1, 1), 0).astype(
        jnp.float32) + 1.0
    eqk = jnp.where((rank3 == kval) & mask3, 1.0, 0.0)
    eq1 = jnp.where((rank3 == 1.0) & mask3, 1.0, 0.0)
    valid = kval <= count.reshape(1, S, 1)
    sel = jnp.where(valid, eqk, eq1)
    return sel.reshape(K * S, n)


def _mlp(h, layers):
    for (W, g, b) in layers:
        h = jnp.maximum(
            g * jnp.dot(W, h, preferred_element_type=jnp.float32) + b, 0.0)
    return h


def _sa(coords, feats_all, npoint, radius, nsample, layers, n):
    K = min(nsample, n)
    new_xyz, nxT = _fps(coords, npoint, n)
    selT = _ball_sel(coords, nxT, radius ** 2, npoint, K, n)
    grouped = jax.lax.dot_general(
        feats_all, selT, (((1,), (1,)), ((), ())),
        preferred_element_type=jnp.float32)
    centers = jnp.concatenate([new_xyz] * K, axis=1)
    h = jnp.concatenate([grouped[0:3] - centers, grouped[3:]], axis=0)
    h = _mlp(h, layers)
    out = h[:, 0:npoint]
    for k in range(1, K):
        out = jnp.maximum(out, h[:, k * npoint:(k + 1) * npoint])
    return new_xyz, nxT, out


def _fp(coords1, c2T, points1, points2, layers, n, m):
    d2 = ((c2T[:, 0:1] - coords1[0:1, :]) ** 2
          + (c2T[:, 1:2] - coords1[1:2, :]) ** 2
          + (c2T[:, 2:3] - coords1[2:3, :]) ** 2)
    rowi = jax.lax.broadcasted_iota(jnp.int32, (m, n), 0)
    dwork = d2
    mns, ohs = [], []
    for _ in range(3):
        mn = jnp.min(dwork, axis=0, keepdims=True)
        selidx = jnp.min(jnp.where(dwork == mn, rowi, m), axis=0,
                         keepdims=True)
        oh = rowi == selidx
        mns.append(mn)
        ohs.append(oh)
        dwork = jnp.where(oh, jnp.float32(jnp.inf), dwork)
    w = [1.0 / (mn + 1e-8) for mn in mns]
    wsum = (w[0] + w[1]) + w[2]
    wm = jnp.zeros((m, n), jnp.float32)
    for j in range(3):
        wm = wm + jnp.where(ohs[j], w[j] / wsum, 0.0)
    interp = jnp.dot(points2, wm, preferred_element_type=jnp.float32)
    h = interp if points1 is None else jnp.concatenate([points1, interp],
                                                       axis=0)
    return _mlp(h, layers)


_LAYOUT = (('sa1', 3), ('sa2', 3), ('sa3', 3), ('fp3', 2), ('fp2', 2),
           ('fp1', 2))


def _body(*refs):
    x_ref = refs[0]
    o_ref = refs[-1]
    pr = refs[1:-1]
    mods = {}
    idx = 0
    for name, nl in _LAYOUT:
        layers = []
        for _ in range(nl):
            layers.append((pr[idx][...], pr[idx + 1][...], pr[idx + 2][...]))
            idx += 3
        mods[name] = layers

    x = x_ref[0]
    l0_xyz = x[0:3, :]
    l1_xyz, l1_xyzT, l1_p = _sa(l0_xyz, x, 16, 0.2, 16, mods['sa1'], 4096)
    f1 = jnp.concatenate([l1_xyz, l1_p], axis=0)
    l2_xyz, l2_xyzT, l2_p = _sa(l1_xyz, f1, 12, 0.4, 16, mods['sa2'], 16)
    f2 = jnp.concatenate([l2_xyz, l2_p], axis=0)
    l3_xyz, l3_xyzT, l3_p = _sa(l2_xyz, f2, 8, 0.8, 16, mods['sa3'], 12)
    l2_p = _fp(l2_xyz, l3_xyzT, l2_p, l3_p, mods['fp3'], 12, 8)
    l1_p = _fp(l1_xyz, l2_xyzT, l1_p, l2_p, mods['fp2'], 16, 12)
    out = _fp(l0_xyz, l1_xyzT, None, l1_p, mods['fp1'], 4096, 16)
    o_ref[0] = out


def kernel(xyz, params):
    B, C, N = xyz.shape
    seq = []
    for name, _ in _LAYOUT:
        for l in params[name]:
            seq.append(l['W'])
            seq.append(l['g'].reshape(-1, 1))
            seq.append(l['b'].reshape(-1, 1))
    in_specs = [pl.BlockSpec((1, C, N), lambda b: (b, 0, 0))]
    for w in seq:
        in_specs.append(
            pl.BlockSpec(w.shape, lambda b, _nd=w.ndim: (0,) * _nd))
    return pl.pallas_call(
        _body,
        grid=(B,),
        in_specs=in_specs,
        out_specs=pl.BlockSpec((1, 256, N), lambda b: (b, 0, 0)),
        out_shape=jax.ShapeDtypeStruct((B, 256, N), jnp.float32),
        compiler_params=pltpu.CompilerParams(
            dimension_semantics=("parallel",)),
    )(xyz, *seq)


# speedup vs baseline: 20.7400x; 2.0638x over previous
import jax
import jax.numpy as jnp
from jax.experimental import pallas as pl
from jax.experimental.pallas import tpu as pltpu


def _fps_batched(X, Y, Z, npoint, n, bsz):
    lane = jax.lax.broadcasted_iota(jnp.int32, (bsz, n), 1)
    scol = jax.lax.broadcasted_iota(jnp.int32, (bsz, npoint), 1)

    def body(i, carry):
        dist, far, CX, CY, CZ = carry
        onehot = (lane == far).astype(jnp.float32)
        cx = jnp.sum(X * onehot, axis=1, keepdims=True)
        cy = jnp.sum(Y * onehot, axis=1, keepdims=True)
        cz = jnp.sum(Z * onehot, axis=1, keepdims=True)
        CX = jnp.where(scol == i, cx, CX)
        CY = jnp.where(scol == i, cy, CY)
        CZ = jnp.where(scol == i, cz, CZ)
        d = (X - cx) ** 2 + (Y - cy) ** 2 + (Z - cz) ** 2
        dist = jnp.minimum(dist, d)
        mx = jnp.max(dist, axis=1, keepdims=True)
        far = jnp.min(jnp.where(dist == mx, lane, n), axis=1, keepdims=True)
        return dist, far, CX, CY, CZ

    dist0 = jnp.full((bsz, n), 1e10, dtype=jnp.float32)
    far0 = jnp.zeros((bsz, 1), dtype=jnp.int32)
    c0 = jnp.zeros((bsz, npoint), dtype=jnp.float32)
    _, _, CX, CY, CZ = jax.lax.fori_loop(0, npoint, body,
                                         (dist0, far0, c0, c0, c0))
    return CX, CY, CZ


def _fps_body(x_ref, l1_ref, l2_ref, l3_ref):
    bsz = x_ref.shape[1]
    n = x_ref.shape[2]
    X = x_ref[0]
    Y = x_ref[1]
    Z = x_ref[2]
    cx1, cy1, cz1 = _fps_batched(X, Y, Z, 16, n, bsz)
    l1_ref[:, 0, :] = cx1
    l1_ref[:, 1, :] = cy1
    l1_ref[:, 2, :] = cz1
    cx2, cy2, cz2 = _fps_batched(cx1, cy1, cz1, 12, 16, bsz)
    l2_ref[:, 0, :] = cx2
    l2_ref[:, 1, :] = cy2
    l2_ref[:, 2, :] = cz2
    cx3, cy3, cz3 = _fps_batched(cx2, cy2, cz2, 8, 12, bsz)
    l3_ref[:, 0, :] = cx3
    l3_ref[:, 1, :] = cy3
    l3_ref[:, 2, :] = cz3


def _ball_sel(coords, nxT, r2, S, K, n):
    d2 = ((nxT[:, 0:1] - coords[0:1, :]) ** 2
          + (nxT[:, 1:2] - coords[1:2, :]) ** 2
          + (nxT[:, 2:3] - coords[2:3, :]) ** 2)
    inmask = jnp.logical_not(d2 > jnp.float32(r2))
    maskf = jnp.where(inmask, 1.0, 0.0)
    rank = maskf
    s = 1
    while s < n:
        rank = rank + jnp.concatenate(
            [jnp.zeros((S, s), jnp.float32), rank[:, :n - s]], axis=1)
        s *= 2
    count = rank[:, n - 1:n]
    rankm3 = (rank * maskf).reshape(1, S, n)
    kval = jax.lax.broadcasted_iota(jnp.int32, (K, 1, 1), 0).astype(
        jnp.float32) + 1.0
    keff = jnp.where(kval <= count.reshape(1, S, 1), kval, 1.0)
    sel = jnp.where(rankm3 == keff, 1.0, 0.0)
    return sel.reshape(K * S, n)


def _mlp(h, layers):
    for (W, g, b) in layers:
        h = jnp.maximum(
            g * jnp.dot(W, h, preferred_element_type=jnp.float32) + b, 0.0)
    return h


def _sa(coords, feats_all, new_xyz, nxT, radius, nsample, layers, n):
    npoint = new_xyz.shape[1]
    K = min(nsample, n)
    selT = _ball_sel(coords, nxT, radius ** 2, npoint, K, n)
    grouped = jax.lax.dot_general(
        feats_all, selT, (((1,), (1,)), ((), ())),
        preferred_element_type=jnp.float32)
    centers = jnp.concatenate([new_xyz] * K, axis=1)
    h = jnp.concatenate([grouped[0:3] - centers, grouped[3:]], axis=0)
    h = _mlp(h, layers)
    out = h[:, 0:npoint]
    for k in range(1, K):
        out = jnp.maximum(out, h[:, k * npoint:(k + 1) * npoint])
    return out


def _fp(coords1, c2T, points1, points2, layers, n, m):
    d2 = ((c2T[:, 0:1] - coords1[0:1, :]) ** 2
          + (c2T[:, 1:2] - coords1[1:2, :]) ** 2
          + (c2T[:, 2:3] - coords1[2:3, :]) ** 2)
    rowi = jax.lax.broadcasted_iota(jnp.int32, (m, n), 0)
    dwork = d2
    mns, ohs = [], []
    for _ in range(3):
        mn = jnp.min(dwork, axis=0, keepdims=True)
        selidx = jnp.min(jnp.where(dwork == mn, rowi, m), axis=0,
                         keepdims=True)
        oh = rowi == selidx
        mns.append(mn)
        ohs.append(oh)
        dwork = jnp.where(oh, jnp.float32(jnp.inf), dwork)
    w = [1.0 / (mn + 1e-8) for mn in mns]
    wsum = (w[0] + w[1]) + w[2]
    wm = jnp.zeros((m, n), jnp.float32)
    for j in range(3):
        wm = wm + jnp.where(ohs[j], w[j] / wsum, 0.0)
    interp = jnp.dot(points2, wm, preferred_element_type=jnp.float32)
    h = interp if points1 is None else jnp.concatenate([points1, interp],
                                                       axis=0)
    return _mlp(h, layers)


_LAYOUT = (('sa1', 3), ('sa2', 3), ('sa3', 3), ('fp3', 2), ('fp2', 2),
           ('fp1', 2))


def _main_body(*refs):
    x_ref, l1_ref, l2_ref, l3_ref = refs[0:4]
    o_ref = refs[-1]
    pr = refs[4:-1]
    mods = {}
    idx = 0
    for name, nl in _LAYOUT:
        layers = []
        for _ in range(nl):
            layers.append((pr[idx][...], pr[idx + 1][...], pr[idx + 2][...]))
            idx += 3
        mods[name] = layers

    x = x_ref[0]
    l0_xyz = x[0:3, :]
    l1_xyz = l1_ref[0]
    l2_xyz = l2_ref[0]
    l3_xyz = l3_ref[0]
    l1_xyzT = jnp.transpose(l1_xyz)
    l2_xyzT = jnp.transpose(l2_xyz)
    l3_xyzT = jnp.transpose(l3_xyz)

    l1_p = _sa(l0_xyz, x, l1_xyz, l1_xyzT, 0.2, 16, mods['sa1'], 4096)
    f1 = jnp.concatenate([l1_xyz, l1_p], axis=0)
    l2_p = _sa(l1_xyz, f1, l2_xyz, l2_xyzT, 0.4, 16, mods['sa2'], 16)
    f2 = jnp.concatenate([l2_xyz, l2_p], axis=0)
    l3_p = _sa(l2_xyz, f2, l3_xyz, l3_xyzT, 0.8, 16, mods['sa3'], 12)
    l2_p = _fp(l2_xyz, l3_xyzT, l2_p, l3_p, mods['fp3'], 12, 8)
    l1_p = _fp(l1_xyz, l2_xyzT, l1_p, l2_p, mods['fp2'], 16, 12)
    out = _fp(l0_xyz, l1_xyzT, None, l1_p, mods['fp1'], 4096, 16)
    o_ref[0] = out


def kernel(xyz, params):
    B, C, N = xyz.shape
    xyz_t = jnp.transpose(xyz, (1, 0, 2))
    l1c, l2c, l3c = pl.pallas_call(
        _fps_body,
        in_specs=[pl.BlockSpec((3, B, N), lambda: (0, 0, 0))],
        out_specs=[pl.BlockSpec((B, 3, 16), lambda: (0, 0, 0)),
                   pl.BlockSpec((B, 3, 12), lambda: (0, 0, 0)),
                   pl.BlockSpec((B, 3, 8), lambda: (0, 0, 0))],
        out_shape=[jax.ShapeDtypeStruct((B, 3, 16), jnp.float32),
                   jax.ShapeDtypeStruct((B, 3, 12), jnp.float32),
                   jax.ShapeDtypeStruct((B, 3, 8), jnp.float32)],
    )(xyz_t[0:3])

    seq = []
    for name, _ in _LAYOUT:
        for l in params[name]:
            seq.append(l['W'])
            seq.append(l['g'].reshape(-1, 1))
            seq.append(l['b'].reshape(-1, 1))
    in_specs = [
        pl.BlockSpec((1, C, N), lambda b: (b, 0, 0)),
        pl.BlockSpec((1, 3, 16), lambda b: (b, 0, 0)),
        pl.BlockSpec((1, 3, 12), lambda b: (b, 0, 0)),
        pl.BlockSpec((1, 3, 8), lambda b: (b, 0, 0)),
    ]
    for w in seq:
        in_specs.append(
            pl.BlockSpec(w.shape, lambda b, _nd=w.ndim: (0,) * _nd))
    return pl.pallas_call(
        _main_body,
        grid=(B,),
        in_specs=in_specs,
        out_specs=pl.BlockSpec((1, 256, N), lambda b: (b, 0, 0)),
        out_shape=jax.ShapeDtypeStruct((B, 256, N), jnp.float32),
        compiler_params=pltpu.CompilerParams(
            dimension_semantics=("parallel",)),
    )(xyz, l1c, l2c, l3c, *seq)
